# baseline (speedup 1.0000x reference)
import functools

import jax
import jax.numpy as jnp
from jax import lax
from jax.experimental import pallas as pl
from jax.experimental.pallas import tpu as pltpu
from jax.experimental.pallas import tpu_sc as plsc

BATCH = 4096
SEQ = 200
DIM = 128
SPLIT0 = 128
SPLIT1 = SEQ - SPLIT0
NUM_CORES = 2
NUM_SUBCORES = 16
NW = NUM_CORES * NUM_SUBCORES
S_PER = BATCH // NW
LANES = 16
DCH = DIM // LANES
NCLS = 2
TC_BLOCK = 512


def _make_sc_pool_kernel():
    mesh = plsc.VectorSubcoreMesh(core_axis_name="c", subcore_axis_name="s")

    @functools.partial(
        pl.kernel,
        mesh=mesh,
        out_type=jax.ShapeDtypeStruct((BATCH, DIM), jnp.float32),
        compiler_params=pltpu.CompilerParams(
            needs_layout_passes=False,
            skip_device_barrier=True,
            disable_bounds_checks=True,
            disable_semaphore_checks=True,
        ),
        scratch_types=[
            pltpu.VMEM((S_PER * SEQ,), jnp.int32),
            pltpu.VMEM((SEQ, DIM), jnp.float32),
            pltpu.VMEM((SEQ, DIM), jnp.float32),
            pltpu.VMEM((S_PER, DIM), jnp.float32),
            pltpu.SemaphoreType.DMA,
            pltpu.SemaphoreType.DMA,
        ],
    )
    def body(idx_hbm, tab_hbm, out_hbm,
             idx_v, rows0, rows1, pooled_v, sem0, sem1):
        wid = lax.axis_index("s") * NUM_CORES + lax.axis_index("c")
        base = wid * (S_PER * SEQ)
        pltpu.sync_copy(idx_hbm.at[pl.ds(base, S_PER * SEQ)], idx_v)

        rows = (rows0, rows1)
        sems = (sem0, sem1)

        def fire(s, k):
            off = s * SEQ
            pltpu.async_copy(tab_hbm.at[idx_v.at[pl.ds(off, SPLIT0)]],
                             rows[k].at[pl.ds(0, SPLIT0)], sems[k])
            pltpu.async_copy(tab_hbm.at[idx_v.at[pl.ds(off + SPLIT0, SPLIT1)]],
                             rows[k].at[pl.ds(SPLIT0, SPLIT1)], sems[k])

        def drain(s, k):
            off = s * SEQ
            pltpu.make_async_copy(tab_hbm.at[idx_v.at[pl.ds(off, SPLIT0)]],
                                  rows[k].at[pl.ds(0, SPLIT0)], sems[k]).wait()
            pltpu.make_async_copy(tab_hbm.at[idx_v.at[pl.ds(off + SPLIT0, SPLIT1)]],
                                  rows[k].at[pl.ds(SPLIT0, SPLIT1)], sems[k]).wait()

        def accum_and_store(s, k):
            rb = rows[k]

            def rbody(r, accs):
                return tuple(accs[d] + rb[r, pl.ds(d * LANES, LANES)]
                             for d in range(DCH))

            accs = lax.fori_loop(
                0, SEQ, rbody,
                tuple(jnp.zeros((LANES,), jnp.float32) for _ in range(DCH)))
            for d in range(DCH):
                pooled_v[s, pl.ds(d * LANES, LANES)] = accs[d]

        fire(0, 0)

        def step(g, carry):
            for k in (0, 1):
                s = 2 * g + k
                drain(s, k)

                @pl.when(s + 1 < S_PER)
                def _():
                    fire(s + 1, 1 - k)

                accum_and_store(s, k)
            return carry

        lax.fori_loop(0, S_PER // 2, step, 0)

        pltpu.sync_copy(pooled_v, out_hbm.at[pl.ds(wid * S_PER, S_PER)])

    return body


_sc_pool = _make_sc_pool_kernel()


def _tc_linear_body(p_ref, w_ref, b_ref, o_ref):
    o_ref[...] = (
        lax.dot_general(p_ref[...], w_ref[...], (((1,), (0,)), ((), ())),
                        preferred_element_type=jnp.float32) * (1.0 / SEQ)
        + b_ref[...]
    )


_tc_linear = pl.pallas_call(
    _tc_linear_body,
    grid=(BATCH // TC_BLOCK,),
    in_specs=[
        pl.BlockSpec((TC_BLOCK, DIM), lambda i: (i, 0)),
        pl.BlockSpec((DIM, NCLS), lambda i: (0, 0)),
        pl.BlockSpec((1, NCLS), lambda i: (0, 0)),
    ],
    out_specs=pl.BlockSpec((TC_BLOCK, NCLS), lambda i: (i, 0)),
    out_shape=jax.ShapeDtypeStruct((BATCH, NCLS), jnp.float32),
)


def kernel(text_indices, emb_table, fc_w, fc_b):
    idx = text_indices.astype(jnp.int32).reshape(BATCH * SEQ)
    pooled = _sc_pool(idx, emb_table)
    wt = fc_w.astype(jnp.float32).T
    return _tc_linear(pooled, wt, fc_b.astype(jnp.float32).reshape(1, NCLS))


# speedup vs baseline: 1.4821x; 1.4821x over previous
import functools

import jax
import jax.numpy as jnp
from jax import lax
from jax.experimental import pallas as pl
from jax.experimental.pallas import tpu as pltpu
from jax.experimental.pallas import tpu_sc as plsc

BATCH = 4096
SEQ = 200
DIM = 128
SPLIT0 = 128
SPLIT1 = SEQ - SPLIT0
NUM_CORES = 2
NUM_SUBCORES = 16
NW = NUM_CORES * NUM_SUBCORES
S_PER = BATCH // NW
LANES = 16
DCH = DIM // LANES
NCLS = 2
TC_BLOCK = 512


def _make_sc_pool_kernel():
    mesh = plsc.VectorSubcoreMesh(core_axis_name="c", subcore_axis_name="s")

    @functools.partial(
        pl.kernel,
        mesh=mesh,
        out_type=jax.ShapeDtypeStruct((BATCH, DIM), jnp.float32),
        compiler_params=pltpu.CompilerParams(
            needs_layout_passes=False,
            skip_device_barrier=True,
            disable_bounds_checks=True,
            disable_semaphore_checks=True,
        ),
        scratch_types=[
            pltpu.VMEM((S_PER * SEQ,), jnp.int32),
            pltpu.VMEM((SEQ, DIM), jnp.float32),
            pltpu.VMEM((SEQ, DIM), jnp.float32),
            pltpu.VMEM((SEQ, DIM), jnp.float32),
            pltpu.VMEM((S_PER, DIM), jnp.float32),
            pltpu.SemaphoreType.DMA,
            pltpu.SemaphoreType.DMA,
            pltpu.SemaphoreType.DMA,
        ],
    )
    def body(idx_hbm, tab_hbm, out_hbm,
             idx_v, rows0, rows1, rows2, pooled_v, sem0, sem1, sem2):
        wid = lax.axis_index("s") * NUM_CORES + lax.axis_index("c")
        base = wid * (S_PER * SEQ)
        pltpu.sync_copy(idx_hbm.at[pl.ds(base, S_PER * SEQ)], idx_v)

        rows = (rows0, rows1, rows2)
        sems = (sem0, sem1, sem2)

        def fire(s, k):
            off = s * SEQ
            pltpu.async_copy(tab_hbm.at[idx_v.at[pl.ds(off, SPLIT0)]],
                             rows[k].at[pl.ds(0, SPLIT0)], sems[k])
            pltpu.async_copy(tab_hbm.at[idx_v.at[pl.ds(off + SPLIT0, SPLIT1)]],
                             rows[k].at[pl.ds(SPLIT0, SPLIT1)], sems[k])

        def drain(s, k):
            off = s * SEQ
            pltpu.make_async_copy(tab_hbm.at[idx_v.at[pl.ds(off, SPLIT0)]],
                                  rows[k].at[pl.ds(0, SPLIT0)], sems[k]).wait()
            pltpu.make_async_copy(tab_hbm.at[idx_v.at[pl.ds(off + SPLIT0, SPLIT1)]],
                                  rows[k].at[pl.ds(SPLIT0, SPLIT1)], sems[k]).wait()

        def accum_and_store(s, k):
            rb = rows[k]

            def rbody(r, accs):
                return tuple(accs[d] + rb[r, pl.ds(d * LANES, LANES)]
                             for d in range(DCH))

            accs = lax.fori_loop(
                0, SEQ, rbody,
                tuple(jnp.zeros((LANES,), jnp.float32) for _ in range(DCH)))
            for d in range(DCH):
                pooled_v[s, pl.ds(d * LANES, LANES)] = accs[d]

        fire(0, 0)
        fire(1, 1)

        def step(g, carry):
            for k in (0, 1, 2):
                s = 3 * g + k
                drain(s, k)

                @pl.when(s + 2 < S_PER)
                def _():
                    fire(s + 2, (k + 2) % 3)

                accum_and_store(s, k)
            return carry

        lax.fori_loop(0, S_PER // 3, step, 0)
        for s, k in ((S_PER - 2, (S_PER - 2) % 3), (S_PER - 1, (S_PER - 1) % 3)):
            drain(s, k)
            accum_and_store(s, k)

        pltpu.sync_copy(pooled_v, out_hbm.at[pl.ds(wid * S_PER, S_PER)])

    return body


_sc_pool = _make_sc_pool_kernel()


def _tc_linear_body(p_ref, w_ref, b_ref, o_ref):
    o_ref[...] = (
        lax.dot_general(p_ref[...], w_ref[...], (((1,), (0,)), ((), ())),
                        preferred_element_type=jnp.float32) * (1.0 / SEQ)
        + b_ref[...]
    )


_tc_linear = pl.pallas_call(
    _tc_linear_body,
    grid=(BATCH // TC_BLOCK,),
    in_specs=[
        pl.BlockSpec((TC_BLOCK, DIM), lambda i: (i, 0)),
        pl.BlockSpec((DIM, NCLS), lambda i: (0, 0)),
        pl.BlockSpec((1, NCLS), lambda i: (0, 0)),
    ],
    out_specs=pl.BlockSpec((TC_BLOCK, NCLS), lambda i: (i, 0)),
    out_shape=jax.ShapeDtypeStruct((BATCH, NCLS), jnp.float32),
)


def kernel(text_indices, emb_table, fc_w, fc_b):
    idx = text_indices.astype(jnp.int32).reshape(BATCH * SEQ)
    pooled = _sc_pool(idx, emb_table)
    wt = fc_w.astype(jnp.float32).T
    return _tc_linear(pooled, wt, fc_b.astype(jnp.float32).reshape(1, NCLS))


# speedup vs baseline: 1.4870x; 1.0033x over previous
import functools

import jax
import jax.numpy as jnp
from jax import lax
from jax.experimental import pallas as pl
from jax.experimental.pallas import tpu as pltpu
from jax.experimental.pallas import tpu_sc as plsc

BATCH = 4096
SEQ = 200
DIM = 128
SPLIT0 = 128
SPLIT1 = SEQ - SPLIT0
NUM_CORES = 2
NUM_SUBCORES = 16
NW = NUM_CORES * NUM_SUBCORES
S_PER = BATCH // NW
LANES = 16
DCH = DIM // LANES
NCLS = 2
NBUF = 4
TC_BLOCK = 512


def _make_sc_pool_kernel():
    mesh = plsc.VectorSubcoreMesh(core_axis_name="c", subcore_axis_name="s")

    @functools.partial(
        pl.kernel,
        mesh=mesh,
        out_type=jax.ShapeDtypeStruct((BATCH, 1, DIM), jnp.float32),
        compiler_params=pltpu.CompilerParams(
            needs_layout_passes=False,
            skip_device_barrier=True,
            disable_bounds_checks=True,
            disable_semaphore_checks=True,
        ),
        scratch_types=[
            pltpu.VMEM((S_PER * SEQ,), jnp.int32),
            pltpu.VMEM((SEQ, DIM), jnp.float32),
            pltpu.VMEM((SEQ, DIM), jnp.float32),
            pltpu.VMEM((SEQ, DIM), jnp.float32),
            pltpu.VMEM((SEQ, DIM), jnp.float32),
            pltpu.VMEM((NBUF, 1, DIM), jnp.float32),
            pltpu.SemaphoreType.DMA,
            pltpu.SemaphoreType.DMA,
            pltpu.SemaphoreType.DMA,
            pltpu.SemaphoreType.DMA,
            pltpu.SemaphoreType.DMA,
            pltpu.SemaphoreType.DMA,
            pltpu.SemaphoreType.DMA,
            pltpu.SemaphoreType.DMA,
        ],
    )
    def body(idx_hbm, tab_hbm, out_hbm,
             idx_v, rows0, rows1, rows2, rows3, stage_v,
             gsem0, gsem1, gsem2, gsem3, osem0, osem1, osem2, osem3):
        wid = lax.axis_index("s") * NUM_CORES + lax.axis_index("c")
        base = wid * (S_PER * SEQ)
        pltpu.sync_copy(idx_hbm.at[pl.ds(base, S_PER * SEQ)], idx_v)

        rows = (rows0, rows1, rows2, rows3)
        gsems = (gsem0, gsem1, gsem2, gsem3)
        osems = (osem0, osem1, osem2, osem3)

        def fire(s, k):
            off = s * SEQ
            pltpu.async_copy(tab_hbm.at[idx_v.at[pl.ds(off, SPLIT0)]],
                             rows[k].at[pl.ds(0, SPLIT0)], gsems[k])
            pltpu.async_copy(tab_hbm.at[idx_v.at[pl.ds(off + SPLIT0, SPLIT1)]],
                             rows[k].at[pl.ds(SPLIT0, SPLIT1)], gsems[k])

        def drain(s, k):
            off = s * SEQ
            pltpu.make_async_copy(tab_hbm.at[idx_v.at[pl.ds(off, SPLIT0)]],
                                  rows[k].at[pl.ds(0, SPLIT0)], gsems[k]).wait()
            pltpu.make_async_copy(tab_hbm.at[idx_v.at[pl.ds(off + SPLIT0, SPLIT1)]],
                                  rows[k].at[pl.ds(SPLIT0, SPLIT1)], gsems[k]).wait()

        def out_slot_descr(k):
            return pltpu.make_async_copy(stage_v.at[pl.ds(k, 1)],
                                         out_hbm.at[pl.ds(0, 1)], osems[k])

        def accum_and_store(s, k, first_round):
            rb = rows[k]

            def rbody(r, accs):
                return tuple(accs[d] + rb[r, pl.ds(d * LANES, LANES)]
                             for d in range(DCH))

            accs = lax.fori_loop(
                0, SEQ, rbody,
                tuple(jnp.zeros((LANES,), jnp.float32) for _ in range(DCH)))

            if not first_round:
                out_slot_descr(k).wait()
            for d in range(DCH):
                stage_v[k, 0, pl.ds(d * LANES, LANES)] = accs[d]
            pltpu.async_copy(stage_v.at[pl.ds(k, 1)],
                             out_hbm.at[pl.ds(wid * S_PER + s, 1)], osems[k])

        for k in range(NBUF - 1):
            fire(k, k)

        def step(g, carry):
            for k in range(NBUF):
                s = NBUF * g + k
                drain(s, k)

                @pl.when(s + NBUF - 1 < S_PER)
                def _():
                    fire(s + NBUF - 1, (k + NBUF - 1) % NBUF)

                accum_and_store(s, k, first_round=False)
            return carry

        for k in range(NBUF):
            drain(k, k)

            @pl.when(k + NBUF - 1 < S_PER)
            def _():
                fire(k + NBUF - 1, (k + NBUF - 1) % NBUF)

            accum_and_store(k, k, first_round=True)

        lax.fori_loop(1, S_PER // NBUF, step, 0)

        for k in range(NBUF):
            out_slot_descr(k).wait()

    return body


_sc_pool = _make_sc_pool_kernel()


def _tc_linear_body(p_ref, w_ref, b_ref, o_ref):
    o_ref[...] = (
        lax.dot_general(p_ref[...], w_ref[...], (((1,), (0,)), ((), ())),
                        preferred_element_type=jnp.float32) * (1.0 / SEQ)
        + b_ref[...]
    )


_tc_linear = pl.pallas_call(
    _tc_linear_body,
    grid=(BATCH // TC_BLOCK,),
    in_specs=[
        pl.BlockSpec((TC_BLOCK, DIM), lambda i: (i, 0)),
        pl.BlockSpec((DIM, NCLS), lambda i: (0, 0)),
        pl.BlockSpec((1, NCLS), lambda i: (0, 0)),
    ],
    out_specs=pl.BlockSpec((TC_BLOCK, NCLS), lambda i: (i, 0)),
    out_shape=jax.ShapeDtypeStruct((BATCH, NCLS), jnp.float32),
)


def kernel(text_indices, emb_table, fc_w, fc_b):
    idx = text_indices.astype(jnp.int32).reshape(BATCH * SEQ)
    pooled = _sc_pool(idx, emb_table).reshape(BATCH, DIM)
    wt = fc_w.astype(jnp.float32).T
    return _tc_linear(pooled, wt, fc_b.astype(jnp.float32).reshape(1, NCLS))
